# baseline (speedup 1.0000x reference)
import functools

import jax
import jax.numpy as jnp
from jax import lax
from jax.experimental import pallas as pl
from jax.experimental.pallas import tpu as pltpu
from jax.experimental.pallas import tpu_sc as plsc

B, C, H, W = 64, 2, 512, 512
M = 128
NPTS = B * M
NW = 16
PTS_W = NPTS // NW
LANES = 16
NVEC = PTS_W // LANES
IDX_ROW = 128
ROWS = PTS_W // IDX_ROW
PLANE = H * W
BSTRIDE = C * PLANE


def _sc_loss(pred_flat, y, x, t0, t1, m):
    mesh = plsc.VectorSubcoreMesh(
        core_axis_name="c", subcore_axis_name="s", num_cores=1)

    @functools.partial(
        pl.kernel,
        mesh=mesh,
        out_type=(
            jax.ShapeDtypeStruct((NW, 2 * LANES), jnp.float32),
            jax.ShapeDtypeStruct((LANES,), jnp.float32),
        ),
        scratch_types=[
            pltpu.VMEM((PTS_W,), jnp.int32),
            pltpu.VMEM((PTS_W,), jnp.int32),
            pltpu.VMEM((PTS_W,), jnp.int32),
            pltpu.VMEM((PTS_W,), jnp.float32),
            pltpu.VMEM((PTS_W,), jnp.float32),
            pltpu.VMEM((2 * ROWS, IDX_ROW), jnp.int32),
            pltpu.VMEM((2 * ROWS, IDX_ROW), jnp.float32),
            pltpu.VMEM((2 * LANES,), jnp.float32),
            pltpu.VMEM((NW, 2 * LANES), jnp.float32),
            pltpu.VMEM((LANES,), jnp.float32),
            pltpu.SemaphoreType.DMA,
        ],
    )
    def body(pred_hbm, y_hbm, x_hbm, t0_hbm, t1_hbm, m_hbm,
             part_hbm, out_hbm,
             y_v, x_v, m_v, t0_v, t1_v, idx_v, g_v, pair_v, red_v, out_v,
             sem):
        wid = lax.axis_index("s")
        base = wid * PTS_W
        pltpu.sync_copy(y_hbm.at[pl.ds(base, PTS_W)], y_v)
        pltpu.sync_copy(x_hbm.at[pl.ds(base, PTS_W)], x_v)
        pltpu.sync_copy(m_hbm.at[pl.ds(base, PTS_W)], m_v)
        pltpu.sync_copy(t0_hbm.at[pl.ds(base, PTS_W)], t0_v)
        pltpu.sync_copy(t1_hbm.at[pl.ds(base, PTS_W)], t1_v)

        for j in range(NVEC):
            yv = y_v[pl.ds(j * LANES, LANES)]
            xv = x_v[pl.ds(j * LANES, LANES)]
            yv = jnp.minimum(jnp.maximum(yv, 0), H - 1)
            xv = jnp.minimum(jnp.maximum(xv, 0), W - 1)
            bscal = wid * (PTS_W // M) + (j * LANES) // M
            a0 = bscal * BSTRIDE + yv * W + xv
            r, col = j // 8, (j % 8) * LANES
            idx_v[r, pl.ds(col, LANES)] = a0
            idx_v[ROWS + r, pl.ds(col, LANES)] = a0 + PLANE

        copies = [
            pltpu.async_copy(pred_hbm.at[idx_v.at[r]], g_v.at[r], sem)
            for r in range(2 * ROWS)
        ]
        for cp in copies:
            cp.wait()

        acc_abs = jnp.zeros((LANES,), jnp.float32)
        acc_cnt = jnp.zeros((LANES,), jnp.float32)
        for j in range(NVEC):
            r, col = j // 8, (j % 8) * LANES
            g0 = g_v[r, pl.ds(col, LANES)]
            g1 = g_v[ROWS + r, pl.ds(col, LANES)]
            t0v = t0_v[pl.ds(j * LANES, LANES)]
            t1v = t1_v[pl.ds(j * LANES, LANES)]
            mf = m_v[pl.ds(j * LANES, LANES)].astype(jnp.float32)
            acc_abs = acc_abs + (jnp.abs(g0 - t0v) + jnp.abs(g1 - t1v)) * mf
            acc_cnt = acc_cnt + mf

        pair_v[pl.ds(0, LANES)] = acc_abs
        pair_v[pl.ds(LANES, LANES)] = acc_cnt
        pltpu.sync_copy(pair_v, part_hbm.at[wid])
        plsc.subcore_barrier()

        @pl.when(wid == 0)
        def _():
            pltpu.sync_copy(part_hbm, red_v)
            s_abs = jnp.zeros((LANES,), jnp.float32)
            s_cnt = jnp.zeros((LANES,), jnp.float32)
            for wdx in range(NW):
                s_abs = s_abs + red_v[wdx, pl.ds(0, LANES)]
                s_cnt = s_cnt + red_v[wdx, pl.ds(LANES, LANES)]
            tot_abs = s_abs[0]
            tot_cnt = s_cnt[0]
            for lane in range(1, LANES):
                tot_abs = tot_abs + s_abs[lane]
                tot_cnt = tot_cnt + s_cnt[lane]
            num_v = jnp.full((LANES,), tot_abs, jnp.float32)
            den_v = jnp.full((LANES,), jnp.maximum(2.0 * tot_cnt, 1.0),
                             jnp.float32)
            out_v[...] = num_v / den_v
            pltpu.sync_copy(out_v, out_hbm)

    return body(pred_flat, y, x, t0, t1, m)


def kernel(pred_offset, target_offset, indices, mask):
    pred_flat = pred_offset.reshape(-1)
    idx = indices.reshape(-1, 2)
    tgt = target_offset.reshape(-1, 2)
    _, out = _sc_loss(pred_flat, idx[:, 0], idx[:, 1],
                      tgt[:, 0], tgt[:, 1], mask.reshape(-1))
    return out[0]


# speedup vs baseline: 2.5100x; 2.5100x over previous
import functools

import jax
import jax.numpy as jnp
from jax import lax
from jax.experimental import pallas as pl
from jax.experimental.pallas import tpu as pltpu
from jax.experimental.pallas import tpu_sc as plsc

B, C, H, W = 64, 2, 512, 512
M = 128
NPTS = B * M
NROW = B * C * H
NW = 16
PTS_W = NPTS // NW
LANES = 16
NCHUNK = PTS_W // LANES
CROWS = 2 * LANES
NBUF = 4
NGRP = NCHUNK // NBUF


def _sc_loss(view, y, x, t0, t1, m):
    mesh = plsc.VectorSubcoreMesh(
        core_axis_name="c", subcore_axis_name="s", num_cores=1)

    @functools.partial(
        pl.kernel,
        mesh=mesh,
        out_type=(
            jax.ShapeDtypeStruct((NW, 2 * LANES), jnp.float32),
            jax.ShapeDtypeStruct((LANES,), jnp.float32),
        ),
        scratch_types=[
            pltpu.VMEM((PTS_W,), jnp.int32),
            pltpu.VMEM((PTS_W,), jnp.int32),
            pltpu.VMEM((PTS_W,), jnp.int32),
            pltpu.VMEM((PTS_W,), jnp.float32),
            pltpu.VMEM((PTS_W,), jnp.float32),
            pltpu.VMEM((NCHUNK, CROWS), jnp.int32),
            pltpu.VMEM((CROWS, W), jnp.float32),
            pltpu.VMEM((CROWS, W), jnp.float32),
            pltpu.VMEM((CROWS, W), jnp.float32),
            pltpu.VMEM((CROWS, W), jnp.float32),
            pltpu.VMEM((2 * LANES,), jnp.float32),
            pltpu.VMEM((NW, 2 * LANES), jnp.float32),
            pltpu.VMEM((LANES,), jnp.float32),
            pltpu.SemaphoreType.DMA,
            pltpu.SemaphoreType.DMA,
            pltpu.SemaphoreType.DMA,
            pltpu.SemaphoreType.DMA,
        ],
    )
    def body(view_hbm, y_hbm, x_hbm, t0_hbm, t1_hbm, m_hbm,
             part_hbm, out_hbm,
             y_v, x_v, m_v, t0_v, t1_v, idx_v,
             gbuf0, gbuf1, gbuf2, gbuf3,
             pair_v, red_v, out_v, sem0, sem1, sem2, sem3):
        wid = lax.axis_index("s")
        base = wid * PTS_W
        pltpu.sync_copy(y_hbm.at[pl.ds(base, PTS_W)], y_v)
        pltpu.sync_copy(x_hbm.at[pl.ds(base, PTS_W)], x_v)
        pltpu.sync_copy(m_hbm.at[pl.ds(base, PTS_W)], m_v)
        pltpu.sync_copy(t0_hbm.at[pl.ds(base, PTS_W)], t0_v)
        pltpu.sync_copy(t1_hbm.at[pl.ds(base, PTS_W)], t1_v)

        gbufs = [gbuf0, gbuf1, gbuf2, gbuf3]
        sems = [sem0, sem1, sem2, sem3]

        for j in range(NCHUNK):
            yv = y_v[pl.ds(j * LANES, LANES)]
            yv = jnp.minimum(jnp.maximum(yv, 0), H - 1)
            bscal = wid * (PTS_W // M) + (j * LANES) // M
            r0 = bscal * (C * H) + yv
            idx_v[j, pl.ds(0, LANES)] = r0
            idx_v[j, pl.ds(LANES, LANES)] = r0 + H

        for s in range(NBUF):
            pltpu.async_copy(view_hbm.at[idx_v.at[s]], gbufs[s], sems[s])

        iot = jnp.arange(LANES, dtype=jnp.int32)

        def group(g, carry):
            acc_abs, acc_cnt = carry
            for s in range(NBUF):
                jj = g * NBUF + s
                gslot = gbufs[s]
                pltpu.make_async_copy(
                    view_hbm.at[idx_v.at[jj]], gslot, sems[s]).wait()
                g0 = jnp.zeros((LANES,), jnp.float32)
                g1 = jnp.zeros((LANES,), jnp.float32)
                xvec = x_v[pl.ds(jj * LANES, LANES)]
                xvec = jnp.minimum(jnp.maximum(xvec, 0), W - 1)
                for l in range(LANES):
                    xs = xvec[l]
                    xa = pl.multiple_of(xs & (W - LANES), LANES)
                    lane = jnp.full((LANES,), xs & (LANES - 1), jnp.int32)
                    v0 = gslot[l, pl.ds(xa, LANES)]
                    v1 = gslot[LANES + l, pl.ds(xa, LANES)]
                    p0 = v0.at[lane].get(mode="promise_in_bounds")
                    p1 = v1.at[lane].get(mode="promise_in_bounds")
                    g0 = jnp.where(iot == l, p0, g0)
                    g1 = jnp.where(iot == l, p1, g1)
                @pl.when(jj + NBUF < NCHUNK)
                def _():
                    pltpu.async_copy(
                        view_hbm.at[idx_v.at[jj + NBUF]], gslot, sems[s])
                t0v = t0_v[pl.ds(jj * LANES, LANES)]
                t1v = t1_v[pl.ds(jj * LANES, LANES)]
                mf = m_v[pl.ds(jj * LANES, LANES)].astype(jnp.float32)
                acc_abs = acc_abs + (jnp.abs(g0 - t0v)
                                     + jnp.abs(g1 - t1v)) * mf
                acc_cnt = acc_cnt + mf
            return acc_abs, acc_cnt

        acc_abs, acc_cnt = lax.fori_loop(
            0, NGRP, group,
            (jnp.zeros((LANES,), jnp.float32),
             jnp.zeros((LANES,), jnp.float32)))

        pair_v[pl.ds(0, LANES)] = acc_abs
        pair_v[pl.ds(LANES, LANES)] = acc_cnt
        pltpu.sync_copy(pair_v, part_hbm.at[wid])
        plsc.subcore_barrier()

        @pl.when(wid == 0)
        def _():
            pltpu.sync_copy(part_hbm, red_v)
            s_abs = jnp.zeros((LANES,), jnp.float32)
            s_cnt = jnp.zeros((LANES,), jnp.float32)
            for wdx in range(NW):
                s_abs = s_abs + red_v[wdx, pl.ds(0, LANES)]
                s_cnt = s_cnt + red_v[wdx, pl.ds(LANES, LANES)]
            tot_abs = s_abs[0]
            tot_cnt = s_cnt[0]
            for lane in range(1, LANES):
                tot_abs = tot_abs + s_abs[lane]
                tot_cnt = tot_cnt + s_cnt[lane]
            num_v = jnp.full((LANES,), tot_abs, jnp.float32)
            den_v = jnp.full((LANES,), jnp.maximum(2.0 * tot_cnt, 1.0),
                             jnp.float32)
            out_v[...] = num_v / den_v
            pltpu.sync_copy(out_v, out_hbm)

    return body(view, y, x, t0, t1, m)


def kernel(pred_offset, target_offset, indices, mask):
    view = pred_offset.reshape(B * C * H, W)
    idx = indices.reshape(-1, 2)
    tgt = target_offset.reshape(-1, 2)
    _, out = _sc_loss(view, idx[:, 0], idx[:, 1],
                      tgt[:, 0], tgt[:, 1], mask.reshape(-1))
    return out[0]


# speedup vs baseline: 2.8199x; 1.1234x over previous
import functools

import jax
import jax.numpy as jnp
from jax import lax
from jax.experimental import pallas as pl
from jax.experimental.pallas import tpu as pltpu
from jax.experimental.pallas import tpu_sc as plsc

B, C, H, W = 64, 2, 512, 512
M = 128
NPTS = B * M
NROW = B * C * H
NC = 2
NW = 32
PTS_W = NPTS // NW
LANES = 16
NCHUNK = PTS_W // LANES
CROWS = 2 * LANES
NBUF = 4
NGRP = NCHUNK // NBUF


def _sc_loss(view, y, x, t0, t1, m):
    mesh = plsc.VectorSubcoreMesh(
        core_axis_name="c", subcore_axis_name="s", num_cores=NC)

    @functools.partial(
        pl.kernel,
        mesh=mesh,
        out_type=(
            jax.ShapeDtypeStruct((NW, 2 * LANES), jnp.float32),
            jax.ShapeDtypeStruct((NC, LANES), jnp.float32),
        ),
        scratch_types=[
            pltpu.VMEM((PTS_W,), jnp.int32),
            pltpu.VMEM((PTS_W,), jnp.int32),
            pltpu.VMEM((PTS_W,), jnp.int32),
            pltpu.VMEM((PTS_W,), jnp.float32),
            pltpu.VMEM((PTS_W,), jnp.float32),
            pltpu.VMEM((NCHUNK, CROWS), jnp.int32),
            pltpu.VMEM((CROWS, W), jnp.float32),
            pltpu.VMEM((CROWS, W), jnp.float32),
            pltpu.VMEM((CROWS, W), jnp.float32),
            pltpu.VMEM((CROWS, W), jnp.float32),
            pltpu.VMEM((2 * LANES,), jnp.float32),
            pltpu.VMEM((NW // NC, 2 * LANES), jnp.float32),
            pltpu.VMEM((LANES,), jnp.float32),
            pltpu.SemaphoreType.DMA,
            pltpu.SemaphoreType.DMA,
            pltpu.SemaphoreType.DMA,
            pltpu.SemaphoreType.DMA,
        ],
    )
    def body(view_hbm, y_hbm, x_hbm, t0_hbm, t1_hbm, m_hbm,
             part_hbm, out_hbm,
             y_v, x_v, m_v, t0_v, t1_v, idx_v,
             gbuf0, gbuf1, gbuf2, gbuf3,
             pair_v, red_v, out_v, sem0, sem1, sem2, sem3):
        cid = lax.axis_index("c")
        sid = lax.axis_index("s")
        wid = cid * (NW // NC) + sid
        base = wid * PTS_W
        pltpu.sync_copy(y_hbm.at[pl.ds(base, PTS_W)], y_v)
        pltpu.sync_copy(x_hbm.at[pl.ds(base, PTS_W)], x_v)
        pltpu.sync_copy(m_hbm.at[pl.ds(base, PTS_W)], m_v)
        pltpu.sync_copy(t0_hbm.at[pl.ds(base, PTS_W)], t0_v)
        pltpu.sync_copy(t1_hbm.at[pl.ds(base, PTS_W)], t1_v)

        gbufs = [gbuf0, gbuf1, gbuf2, gbuf3]
        sems = [sem0, sem1, sem2, sem3]

        for j in range(NCHUNK):
            yv = y_v[pl.ds(j * LANES, LANES)]
            yv = jnp.minimum(jnp.maximum(yv, 0), H - 1)
            bscal = wid * (PTS_W // M) + (j * LANES) // M
            r0 = bscal * (C * H) + yv
            idx_v[j, pl.ds(0, LANES)] = r0
            idx_v[j, pl.ds(LANES, LANES)] = r0 + H

        for s in range(NBUF):
            pltpu.async_copy(view_hbm.at[idx_v.at[s]], gbufs[s], sems[s])

        iot = jnp.arange(LANES, dtype=jnp.int32)

        def group(g, carry):
            acc_abs, acc_cnt = carry
            for s in range(NBUF):
                jj = g * NBUF + s
                gslot = gbufs[s]
                pltpu.make_async_copy(
                    view_hbm.at[idx_v.at[jj]], gslot, sems[s]).wait()
                g0 = jnp.zeros((LANES,), jnp.float32)
                g1 = jnp.zeros((LANES,), jnp.float32)
                xvec = x_v[pl.ds(jj * LANES, LANES)]
                xvec = jnp.minimum(jnp.maximum(xvec, 0), W - 1)
                for l in range(LANES):
                    xs = xvec[l]
                    xa = pl.multiple_of(xs & (W - LANES), LANES)
                    lane = jnp.full((LANES,), xs & (LANES - 1), jnp.int32)
                    v0 = gslot[l, pl.ds(xa, LANES)]
                    v1 = gslot[LANES + l, pl.ds(xa, LANES)]
                    p0 = v0.at[lane].get(mode="promise_in_bounds")
                    p1 = v1.at[lane].get(mode="promise_in_bounds")
                    g0 = jnp.where(iot == l, p0, g0)
                    g1 = jnp.where(iot == l, p1, g1)
                @pl.when(jj + NBUF < NCHUNK)
                def _():
                    pltpu.async_copy(
                        view_hbm.at[idx_v.at[jj + NBUF]], gslot, sems[s])
                t0v = t0_v[pl.ds(jj * LANES, LANES)]
                t1v = t1_v[pl.ds(jj * LANES, LANES)]
                mf = m_v[pl.ds(jj * LANES, LANES)].astype(jnp.float32)
                acc_abs = acc_abs + (jnp.abs(g0 - t0v)
                                     + jnp.abs(g1 - t1v)) * mf
                acc_cnt = acc_cnt + mf
            return acc_abs, acc_cnt

        acc_abs, acc_cnt = lax.fori_loop(
            0, NGRP, group,
            (jnp.zeros((LANES,), jnp.float32),
             jnp.zeros((LANES,), jnp.float32)))

        pair_v[pl.ds(0, LANES)] = acc_abs
        pair_v[pl.ds(LANES, LANES)] = acc_cnt
        pltpu.sync_copy(pair_v, part_hbm.at[wid])
        plsc.subcore_barrier()

        @pl.when(sid == 0)
        def _():
            pltpu.sync_copy(part_hbm.at[pl.ds(cid * (NW // NC), NW // NC)],
                            red_v)
            s_abs = jnp.zeros((LANES,), jnp.float32)
            s_cnt = jnp.zeros((LANES,), jnp.float32)
            for wdx in range(NW // NC):
                s_abs = s_abs + red_v[wdx, pl.ds(0, LANES)]
                s_cnt = s_cnt + red_v[wdx, pl.ds(LANES, LANES)]
            tot_abs = s_abs[0]
            tot_cnt = s_cnt[0]
            for lane in range(1, LANES):
                tot_abs = tot_abs + s_abs[lane]
                tot_cnt = tot_cnt + s_cnt[lane]
            out_v[...] = jnp.where(
                iot == 0, jnp.full((LANES,), tot_abs, jnp.float32),
                jnp.where(iot == 1, jnp.full((LANES,), tot_cnt, jnp.float32),
                          jnp.zeros((LANES,), jnp.float32)))
            pltpu.sync_copy(out_v, out_hbm.at[cid])

    return body(view, y, x, t0, t1, m)


def kernel(pred_offset, target_offset, indices, mask):
    view = pred_offset.reshape(B * C * H, W)
    idx = indices.reshape(-1, 2)
    tgt = target_offset.reshape(-1, 2)
    _, out = _sc_loss(view, idx[:, 0], idx[:, 1],
                      tgt[:, 0], tgt[:, 1], mask.reshape(-1))
    tot_abs = out[0, 0] + out[1, 0]
    tot_cnt = out[0, 1] + out[1, 1]
    return tot_abs / jnp.maximum(2.0 * tot_cnt, 1.0)
